# Initial kernel scaffold; baseline (speedup 1.0000x reference)
import jax
import jax.numpy as jnp
from jax.experimental import pallas as pl


def kernel(x, pos, batch, lin01_w, bn1_g, bn1_b, c1_W, c1_as, c1_ad, c1_b, c2_W, c2_as, c2_ad, c2_b, Wih0, Whh0, bih0, bhh0, Wih1, Whh1, bih1, bhh1, lin2_w, bn2_g, bn2_b, lin3_w, lin3_b):
    raise NotImplementedError("write your pallas kernel here")


# speedup vs baseline: 5.2350x; 5.2350x over previous
import functools
import math

import jax
import jax.numpy as jnp
from jax.experimental import pallas as pl
from jax.experimental.pallas import tpu as pltpu

_N = 10000
_H = 8
_C = 64
_NB = 16
_R2 = 16.0
_NEG = -1e30
_BN_SCALE = 1.0 / math.sqrt(1.1)

_BM = 256
_BS = 512
_MP = 10240
_NDB = _MP // _BM
_NSB = _MP // _BS
_BSA = 512


def _mm1_body(x_ref, w_ref, g_ref, b_ref, o_ref):
    k = pl.program_id(2)
    nk = pl.num_programs(2)
    part = jax.lax.dot_general(
        x_ref[...], w_ref[...], (((1,), (1,)), ((), ())),
        preferred_element_type=jnp.float32)

    @pl.when(k == 0)
    def _():
        o_ref[...] = part

    @pl.when(k > 0)
    def _():
        o_ref[...] = o_ref[...] + part

    @pl.when(k == nk - 1)
    def _():
        y = jnp.maximum(o_ref[...], 0.0) * _BN_SCALE
        o_ref[...] = y * g_ref[...] + b_ref[...]


def _mm1(x_pad, w, g, b):
    bm, bn, bk = 512, 1024, 512
    grid = (_MP // bm, 2048 // bn, 2048 // bk)
    return pl.pallas_call(
        _mm1_body,
        grid=grid,
        in_specs=[
            pl.BlockSpec((bm, bk), lambda i, j, k: (i, k)),
            pl.BlockSpec((bn, bk), lambda i, j, k: (j, k)),
            pl.BlockSpec((1, bn), lambda i, j, k: (0, j)),
            pl.BlockSpec((1, bn), lambda i, j, k: (0, j)),
        ],
        out_specs=pl.BlockSpec((bm, bn), lambda i, j, k: (i, j)),
        out_shape=jax.ShapeDtypeStruct((_MP, 2048), jnp.float32),
        compiler_params=pltpu.CompilerParams(
            dimension_semantics=("parallel", "parallel", "arbitrary")),
    )(x_pad, w, g.reshape(1, -1), b.reshape(1, -1))


def _proj_body(h_ref, w_ref, a_ref, hp_ref, es_ref):
    k = pl.program_id(1)
    nk = pl.num_programs(1)
    part = jax.lax.dot_general(
        h_ref[...], w_ref[...], (((1,), (0,)), ((), ())),
        preferred_element_type=jnp.float32)

    @pl.when(k == 0)
    def _():
        hp_ref[...] = part

    @pl.when(k > 0)
    def _():
        hp_ref[...] = hp_ref[...] + part

    @pl.when(k == nk - 1)
    def _():
        es_ref[...] = jax.lax.dot_general(
            hp_ref[...], a_ref[...], (((1,), (0,)), ((), ())),
            preferred_element_type=jnp.float32)


def _proj(h, wc, amat):
    bm, bk = 512, 512
    grid = (_MP // bm, 2048 // bk)
    return pl.pallas_call(
        _proj_body,
        grid=grid,
        in_specs=[
            pl.BlockSpec((bm, bk), lambda i, k: (i, k)),
            pl.BlockSpec((bk, 1024), lambda i, k: (k, 0)),
            pl.BlockSpec((1024, 32), lambda i, k: (0, 0)),
        ],
        out_specs=[
            pl.BlockSpec((bm, 1024), lambda i, k: (i, 0)),
            pl.BlockSpec((bm, 32), lambda i, k: (i, 0)),
        ],
        out_shape=[
            jax.ShapeDtypeStruct((_MP, 1024), jnp.float32),
            jax.ShapeDtypeStruct((_MP, 32), jnp.float32),
        ],
        compiler_params=pltpu.CompilerParams(
            dimension_semantics=("parallel", "arbitrary")),
    )(h, wc, amat)


def _gat_body(sb_ref, posd_ref, poss_ref, psqd_ref, psqs_ref, bd_ref, bs_ref,
              esedd_ref, esedt_ref, hp_ref, bias_ref, o_ref,
              m_ref, l_ref, acc_ref):
    i = pl.program_id(0)
    j = pl.program_id(1)
    nj = pl.num_programs(1)
    start = sb_ref[0, i]
    end = sb_ref[1, i]

    @pl.when(j == 0)
    def _():
        m_ref[...] = jnp.full_like(m_ref[...], _NEG)
        l_ref[...] = jnp.zeros_like(l_ref[...])
        acc_ref[...] = jnp.zeros_like(acc_ref[...])

    @pl.when((j >= start) & (j <= end))
    def _():
        pd = posd_ref[...]
        ps = poss_ref[...]
        d2 = psqd_ref[...] + psqs_ref[...] - 2.0 * jax.lax.dot_general(
            pd, ps, (((1,), (1,)), ((), ())),
            preferred_element_type=jnp.float32)
        mask = (d2 < _R2) & (bd_ref[...] == bs_ref[...])
        hp = hp_ref[...]
        for layer in range(2):
            for h in range(_H):
                c = layer * _H + h
                es = esedt_ref[layer * 16 + h:layer * 16 + h + 1, :]
                ed = esedd_ref[:, layer * 16 + 8 + h:layer * 16 + 9 + h]
                al = es + ed
                al = jnp.where(al >= 0.0, al, 0.2 * al)
                alm = jnp.where(mask, al, _NEG)
                bmax = jnp.max(alm, axis=1, keepdims=True)
                m_old = m_ref[:, c:c + 1]
                m_new = jnp.maximum(m_old, bmax)
                safe_old = jnp.where(m_old > -1e29, m_old, 0.0)
                safe_new = jnp.where(m_new > -1e29, m_new, 0.0)
                alpha = jnp.exp(safe_old - safe_new)
                p = jnp.where(mask, jnp.exp(al - safe_new), 0.0)
                l_new = l_ref[:, c:c + 1] * alpha + jnp.sum(
                    p, axis=1, keepdims=True)
                m_ref[:, c:c + 1] = m_new
                l_ref[:, c:c + 1] = l_new
                lo = layer * 512 + h * _C
                pv = jax.lax.dot_general(
                    p, hp[:, lo:lo + _C], (((1,), (0,)), ((), ())),
                    preferred_element_type=jnp.float32)
                acc_ref[:, lo:lo + _C] = acc_ref[:, lo:lo + _C] * alpha + pv

    @pl.when(j == nj - 1)
    def _():
        bias = bias_ref[...]
        for layer in range(2):
            for h in range(_H):
                c = layer * _H + h
                lo = layer * 512 + h * _C
                den = l_ref[:, c:c + 1] + 1e-16
                val = acc_ref[:, lo:lo + _C] / den + bias[:, lo:lo + _C]
                o_ref[:, lo:lo + _C] = jnp.maximum(val, 0.0)


def _gat(sb, pos8, psq_col, psq_row, b_col, b_row, esed, esedt, hp, biasc):
    def _src(i, j, s):
        return jnp.clip(j, s[0, i], s[1, i])

    grid = (_NDB, _NSB)
    return pl.pallas_call(
        _gat_body,
        grid_spec=pltpu.PrefetchScalarGridSpec(
            num_scalar_prefetch=1,
            grid=grid,
            in_specs=[
                pl.BlockSpec((_BM, 8), lambda i, j, s: (i, 0)),
                pl.BlockSpec((_BS, 8), lambda i, j, s: (_src(i, j, s), 0)),
                pl.BlockSpec((_BM, 1), lambda i, j, s: (i, 0)),
                pl.BlockSpec((1, _BS), lambda i, j, s: (0, _src(i, j, s))),
                pl.BlockSpec((_BM, 1), lambda i, j, s: (i, 0)),
                pl.BlockSpec((1, _BS), lambda i, j, s: (0, _src(i, j, s))),
                pl.BlockSpec((_BM, 32), lambda i, j, s: (i, 0)),
                pl.BlockSpec((32, _BS), lambda i, j, s: (0, _src(i, j, s))),
                pl.BlockSpec((_BS, 1024), lambda i, j, s: (_src(i, j, s), 0)),
                pl.BlockSpec((1, 1024), lambda i, j, s: (0, 0)),
            ],
            out_specs=pl.BlockSpec((_BM, 1024), lambda i, j, s: (i, 0)),
            scratch_shapes=[
                pltpu.VMEM((_BM, 16), jnp.float32),
                pltpu.VMEM((_BM, 16), jnp.float32),
                pltpu.VMEM((_BM, 1024), jnp.float32),
            ],
        ),
        out_shape=jax.ShapeDtypeStruct((_MP, 1024), jnp.float32),
        compiler_params=pltpu.CompilerParams(
            dimension_semantics=("arbitrary", "arbitrary")),
    )(sb, pos8, pos8, psq_col, psq_row, b_col, b_row, esed, esedt, hp, biasc)


def _lstm_body(xq_ref, hin_ref, wih_ref, whh_ref, bias_ref, c_ref,
               h_out_ref, c_out_ref, acc_ref, *, nkx, nkh):
    k = pl.program_id(0)

    @pl.when(k == 0)
    def _():
        acc_ref[...] = jnp.zeros_like(acc_ref[...])

    @pl.when(k < nkx)
    def _():
        acc_ref[...] = acc_ref[...] + jax.lax.dot_general(
            xq_ref[...], wih_ref[...], (((1,), (1,)), ((), ())),
            preferred_element_type=jnp.float32)

    @pl.when(k >= nkx)
    def _():
        acc_ref[...] = acc_ref[...] + jax.lax.dot_general(
            hin_ref[...], whh_ref[...], (((1,), (1,)), ((), ())),
            preferred_element_type=jnp.float32)

    @pl.when(k == nkx + nkh - 1)
    def _():
        g = acc_ref[...] + bias_ref[...]
        gi = g[:, 0:1024]
        gf = g[:, 1024:2048]
        gg = g[:, 2048:3072]
        go = g[:, 3072:4096]
        c_new = jax.nn.sigmoid(gf) * c_ref[...] + jax.nn.sigmoid(gi) * jnp.tanh(gg)
        h_out_ref[...] = jax.nn.sigmoid(go) * jnp.tanh(c_new)
        c_out_ref[...] = c_new


def _lstm(xq, hin, wih, whh, bias, c_prev):
    bk = 512
    nkx = xq.shape[1] // bk
    nkh = hin.shape[1] // bk
    grid = (nkx + nkh,)
    return pl.pallas_call(
        functools.partial(_lstm_body, nkx=nkx, nkh=nkh),
        grid=grid,
        in_specs=[
            pl.BlockSpec((16, bk), lambda k: (0, jnp.minimum(k, nkx - 1))),
            pl.BlockSpec((16, bk),
                         lambda k: (0, jnp.clip(k - nkx, 0, nkh - 1))),
            pl.BlockSpec((4096, bk), lambda k: (0, jnp.minimum(k, nkx - 1))),
            pl.BlockSpec((4096, bk),
                         lambda k: (0, jnp.clip(k - nkx, 0, nkh - 1))),
            pl.BlockSpec((1, 4096), lambda k: (0, 0)),
            pl.BlockSpec((16, 1024), lambda k: (0, 0)),
        ],
        out_specs=[
            pl.BlockSpec((16, 1024), lambda k: (0, 0)),
            pl.BlockSpec((16, 1024), lambda k: (0, 0)),
        ],
        out_shape=[
            jax.ShapeDtypeStruct((16, 1024), jnp.float32),
            jax.ShapeDtypeStruct((16, 1024), jnp.float32),
        ],
        scratch_shapes=[pltpu.VMEM((16, 4096), jnp.float32)],
        compiler_params=pltpu.CompilerParams(
            dimension_semantics=("arbitrary",),
            vmem_limit_bytes=100 * 1024 * 1024),
    )(xq, hin, wih, whh, bias, c_prev)


def _att_body(xc_ref, q_ref, b_ref, s_ref, r_ref, e_scr, mx_scr):
    p = pl.program_id(0)
    b = pl.program_id(1)
    oh = b_ref[...] == jax.lax.broadcasted_iota(jnp.int32, (1, _NB), 1)

    @pl.when((p == 0) & (b == 0))
    def _():
        mx_scr[...] = jnp.full_like(mx_scr[...], _NEG)
        s_ref[...] = jnp.zeros_like(s_ref[...])
        r_ref[...] = jnp.zeros_like(r_ref[...])

    @pl.when(p == 0)
    def _():
        e_full = jax.lax.dot_general(
            xc_ref[...], q_ref[...], (((1,), (1,)), ((), ())),
            preferred_element_type=jnp.float32)
        e = jnp.sum(jnp.where(oh, e_full, 0.0), axis=1, keepdims=True)
        e_scr[pl.ds(b * _BSA, _BSA), :] = e
        mx_scr[...] = jnp.maximum(
            mx_scr[...],
            jnp.max(jnp.where(oh, e_full, _NEG), axis=0, keepdims=True))

    @pl.when(p == 1)
    def _():
        e = e_scr[pl.ds(b * _BSA, _BSA), :]
        mxg = jnp.sum(jnp.where(oh, mx_scr[...], 0.0), axis=1, keepdims=True)
        ee = jnp.exp(e - mxg)
        wee = jnp.where(oh, ee, 0.0)
        s_ref[...] = s_ref[...] + jnp.sum(wee, axis=0, keepdims=True)
        r_ref[...] = r_ref[...] + jax.lax.dot_general(
            wee, xc_ref[...], (((0,), (0,)), ((), ())),
            preferred_element_type=jnp.float32)


def _att(xc, q, b_col):
    grid = (2, _MP // _BSA)
    return pl.pallas_call(
        _att_body,
        grid=grid,
        in_specs=[
            pl.BlockSpec((_BSA, 1024), lambda p, b: (b, 0)),
            pl.BlockSpec((16, 1024), lambda p, b: (0, 0)),
            pl.BlockSpec((_BSA, 1), lambda p, b: (b, 0)),
        ],
        out_specs=[
            pl.BlockSpec((1, _NB), lambda p, b: (0, 0)),
            pl.BlockSpec((_NB, 1024), lambda p, b: (0, 0)),
        ],
        out_shape=[
            jax.ShapeDtypeStruct((1, _NB), jnp.float32),
            jax.ShapeDtypeStruct((_NB, 1024), jnp.float32),
        ],
        scratch_shapes=[
            pltpu.VMEM((_MP, 1), jnp.float32),
            pltpu.VMEM((1, _NB), jnp.float32),
        ],
        compiler_params=pltpu.CompilerParams(
            dimension_semantics=("arbitrary", "arbitrary")),
    )(xc, q, b_col)


def _head_body(xs_ref, w2_ref, g2_ref, b2_ref, w3_ref, b3_ref, o_ref,
               acc_ref):
    k = pl.program_id(0)
    nk = pl.num_programs(0)

    @pl.when(k == 0)
    def _():
        acc_ref[...] = jnp.zeros_like(acc_ref[...])

    acc_ref[...] = acc_ref[...] + jax.lax.dot_general(
        xs_ref[...], w2_ref[...], (((1,), (1,)), ((), ())),
        preferred_element_type=jnp.float32)

    @pl.when(k == nk - 1)
    def _():
        xl = jnp.maximum(acc_ref[...], 0.0) * _BN_SCALE
        xl = xl * g2_ref[...] + b2_ref[...]
        o_ref[...] = jax.lax.dot_general(
            xl, w3_ref[...], (((1,), (1,)), ((), ())),
            preferred_element_type=jnp.float32) + b3_ref[...]


def _head(xs, w2, g2, b2, w3, b3):
    bk = 512
    grid = (2048 // bk,)
    return pl.pallas_call(
        _head_body,
        grid=grid,
        in_specs=[
            pl.BlockSpec((16, bk), lambda k: (0, k)),
            pl.BlockSpec((1024, bk), lambda k: (0, k)),
            pl.BlockSpec((1, 1024), lambda k: (0, 0)),
            pl.BlockSpec((1, 1024), lambda k: (0, 0)),
            pl.BlockSpec((10, 1024), lambda k: (0, 0)),
            pl.BlockSpec((1, 10), lambda k: (0, 0)),
        ],
        out_specs=pl.BlockSpec((16, 10), lambda k: (0, 0)),
        out_shape=jax.ShapeDtypeStruct((16, 10), jnp.float32),
        scratch_shapes=[pltpu.VMEM((16, 1024), jnp.float32)],
        compiler_params=pltpu.CompilerParams(
            dimension_semantics=("arbitrary",)),
    )(xs, w2, g2.reshape(1, -1), b2.reshape(1, -1), w3, b3.reshape(1, -1))


def _amat(c1_as, c1_ad, c2_as, c2_ad):
    amat = jnp.zeros((1024, 32), jnp.float32)
    rows = jnp.arange(_H * _C)
    head = rows // _C
    amat = amat.at[rows, head].set(c1_as.reshape(-1))
    amat = amat.at[rows, 8 + head].set(c1_ad.reshape(-1))
    amat = amat.at[512 + rows, 16 + head].set(c2_as.reshape(-1))
    amat = amat.at[512 + rows, 24 + head].set(c2_ad.reshape(-1))
    return amat


@jax.jit
def kernel(x, pos, batch, lin01_w, bn1_g, bn1_b, c1_W, c1_as, c1_ad, c1_b,
           c2_W, c2_as, c2_ad, c2_b, Wih0, Whh0, bih0, bhh0, Wih1, Whh1,
           bih1, bhh1, lin2_w, bn2_g, bn2_b, lin3_w, lin3_b):
    n = x.shape[0]
    pad = _MP - n

    x_pad = jnp.pad(x, ((0, pad), (0, 0)))
    pos8 = jnp.pad(pos.astype(jnp.float32), ((0, pad), (0, 5)))
    batch_pad = jnp.concatenate(
        [batch.astype(jnp.int32), jnp.full((pad,), 127, jnp.int32)])
    psq = jnp.sum(pos8 * pos8, axis=1)
    psq_col = psq.reshape(_MP, 1)
    psq_row = psq.reshape(1, _MP)
    b_col = batch_pad.reshape(_MP, 1)
    b_row = batch_pad.reshape(1, _MP)

    row0 = jnp.arange(_NDB, dtype=jnp.int32) * _BM
    g_lo = batch_pad[row0]
    g_hi = batch_pad[row0 + _BM - 1]
    start_row = jnp.searchsorted(batch_pad, g_lo, side='left')
    end_row = jnp.searchsorted(batch_pad, g_hi, side='right')
    sb = jnp.stack([start_row // _BS, (end_row - 1) // _BS]).astype(jnp.int32)

    h = _mm1(x_pad, lin01_w, bn1_g, bn1_b)

    wc = jnp.concatenate([c1_W, c2_W], axis=1)
    amat = _amat(c1_as, c1_ad, c2_as, c2_ad)
    hp, esed = _proj(h, wc, amat)
    esedt = esed.T

    biasc = jnp.concatenate([c1_b, c2_b]).reshape(1, 1024)
    xc = _gat(sb, pos8, psq_col, psq_row, b_col, b_row, esed, esedt, hp,
              biasc)

    bias0 = (bih0 + bhh0).reshape(1, 4096)
    bias1 = (bih1 + bhh1).reshape(1, 4096)
    q_star = jnp.zeros((_NB, 2048), jnp.float32)
    h0 = jnp.zeros((_NB, 1024), jnp.float32)
    c0 = h0
    h1 = h0
    c1 = h0
    for _ in range(2):
        h0, c0 = _lstm(q_star, h0, Wih0, Whh0, bias0, c0)
        h1, c1 = _lstm(h0, h1, Wih1, Whh1, bias1, c1)
        s, r = _att(xc, h1, b_col)
        r = r / (s.reshape(_NB, 1) + 1e-16)
        q_star = jnp.concatenate([h1, r], axis=1)

    return _head(q_star, lin2_w, bn2_g, bn2_b, lin3_w, lin3_b)


# speedup vs baseline: 6.9560x; 1.3287x over previous
import functools
import math

import jax
import jax.numpy as jnp
from jax.experimental import pallas as pl
from jax.experimental.pallas import tpu as pltpu

_N = 10000
_H = 8
_C = 64
_NB = 16
_R2 = 16.0
_NEG = -1e30
_BN_SCALE = 1.0 / math.sqrt(1.1)

_BM = 256
_BS = 512
_MP = 10240
_NDB = _MP // _BM
_NSB = _MP // _BS
_BSA = 512


def _mm1_body(x_ref, w_ref, g_ref, b_ref, o_ref):
    k = pl.program_id(2)
    nk = pl.num_programs(2)
    part = jax.lax.dot_general(
        x_ref[...], w_ref[...], (((1,), (1,)), ((), ())),
        preferred_element_type=jnp.float32)

    @pl.when(k == 0)
    def _():
        o_ref[...] = part

    @pl.when(k > 0)
    def _():
        o_ref[...] = o_ref[...] + part

    @pl.when(k == nk - 1)
    def _():
        y = jnp.maximum(o_ref[...], 0.0) * _BN_SCALE
        o_ref[...] = y * g_ref[...] + b_ref[...]


def _mm1(x_pad, w, g, b):
    bm, bn, bk = 2048, 1024, 512
    grid = (_MP // bm, 2048 // bn, 2048 // bk)
    return pl.pallas_call(
        _mm1_body,
        grid=grid,
        in_specs=[
            pl.BlockSpec((bm, bk), lambda i, j, k: (i, k)),
            pl.BlockSpec((bn, bk), lambda i, j, k: (j, k)),
            pl.BlockSpec((1, bn), lambda i, j, k: (0, j)),
            pl.BlockSpec((1, bn), lambda i, j, k: (0, j)),
        ],
        out_specs=pl.BlockSpec((bm, bn), lambda i, j, k: (i, j)),
        out_shape=jax.ShapeDtypeStruct((_MP, 2048), jnp.float32),
        compiler_params=pltpu.CompilerParams(
            dimension_semantics=("parallel", "parallel", "arbitrary"),
            vmem_limit_bytes=100 * 1024 * 1024),
    )(x_pad, w, g.reshape(1, -1), b.reshape(1, -1))


def _proj_body(h_ref, w_ref, a_ref, hp_ref, es_ref, em_ref):
    i = pl.program_id(0)
    k = pl.program_id(1)
    nk = pl.num_programs(1)
    part = jax.lax.dot_general(
        h_ref[...], w_ref[...], (((1,), (0,)), ((), ())),
        preferred_element_type=jnp.float32)

    @pl.when(k == 0)
    def _():
        hp_ref[...] = part

    @pl.when(k > 0)
    def _():
        hp_ref[...] = hp_ref[...] + part

    @pl.when(k == nk - 1)
    def _():
        es = jax.lax.dot_general(
            hp_ref[...], a_ref[...], (((1,), (0,)), ((), ())),
            preferred_element_type=jnp.float32)
        es_ref[...] = es
        cmax = jnp.max(es, axis=0, keepdims=True)

        @pl.when(i == 0)
        def _():
            em_ref[...] = cmax

        @pl.when(i > 0)
        def _():
            em_ref[...] = jnp.maximum(em_ref[...], cmax)


def _proj(h, wc, amat):
    bm, bk = 2048, 512
    grid = (_MP // bm, 2048 // bk)
    return pl.pallas_call(
        _proj_body,
        grid=grid,
        in_specs=[
            pl.BlockSpec((bm, bk), lambda i, k: (i, k)),
            pl.BlockSpec((bk, 1024), lambda i, k: (k, 0)),
            pl.BlockSpec((1024, 32), lambda i, k: (0, 0)),
        ],
        out_specs=[
            pl.BlockSpec((bm, 1024), lambda i, k: (i, 0)),
            pl.BlockSpec((bm, 32), lambda i, k: (i, 0)),
            pl.BlockSpec((1, 32), lambda i, k: (0, 0)),
        ],
        out_shape=[
            jax.ShapeDtypeStruct((_MP, 1024), jnp.float32),
            jax.ShapeDtypeStruct((_MP, 32), jnp.float32),
            jax.ShapeDtypeStruct((1, 32), jnp.float32),
        ],
        compiler_params=pltpu.CompilerParams(
            dimension_semantics=("arbitrary", "arbitrary"),
            vmem_limit_bytes=100 * 1024 * 1024),
    )(h, wc, amat)


def _gat_body(sb_ref, posd_ref, poss_ref, psqd_ref, psqs_ref, bd_ref, bs_ref,
              esedd_ref, esedt_ref, esmax_ref, hp_ref, bias_ref, o_ref,
              m_ref, l_ref, acc_ref):
    i = pl.program_id(0)
    j = pl.program_id(1)
    nj = pl.num_programs(1)
    start = sb_ref[0, i]
    end = sb_ref[1, i]

    @pl.when(j == 0)
    def _():
        l_ref[...] = jnp.zeros_like(l_ref[...])
        acc_ref[...] = jnp.zeros_like(acc_ref[...])
        for layer in range(2):
            for h in range(_H):
                c = layer * _H + h
                esm = esmax_ref[0:1, layer * 16 + h:layer * 16 + h + 1]
                ed = esedd_ref[:, layer * 16 + 8 + h:layer * 16 + 9 + h]
                mm = esm + ed
                m_ref[:, c:c + 1] = jnp.maximum(mm, 0.2 * mm)

    @pl.when((j >= start) & (j <= end))
    def _():
        pd = posd_ref[...]
        ps = poss_ref[...]
        d2 = psqd_ref[...] + psqs_ref[...] - 2.0 * jax.lax.dot_general(
            pd, ps, (((1,), (1,)), ((), ())),
            preferred_element_type=jnp.float32)
        mask = (d2 < _R2) & (bd_ref[...] == bs_ref[...])
        mask_add = jnp.where(mask, 0.0, _NEG)
        hp = hp_ref[...]
        for layer in range(2):
            for h in range(_H):
                c = layer * _H + h
                es = esedt_ref[layer * 16 + h:layer * 16 + h + 1, :]
                ed = esedd_ref[:, layer * 16 + 8 + h:layer * 16 + 9 + h]
                al = es + ed
                al = jnp.maximum(al, 0.2 * al)
                p = jnp.exp(al + mask_add - m_ref[:, c:c + 1])
                l_ref[:, c:c + 1] = l_ref[:, c:c + 1] + jnp.sum(
                    p, axis=1, keepdims=True)
                lo = layer * 512 + h * _C
                pv = jax.lax.dot_general(
                    p, hp[:, lo:lo + _C], (((1,), (0,)), ((), ())),
                    preferred_element_type=jnp.float32)
                acc_ref[:, lo:lo + _C] = acc_ref[:, lo:lo + _C] + pv

    @pl.when(j == nj - 1)
    def _():
        bias = bias_ref[...]
        for layer in range(2):
            for h in range(_H):
                c = layer * _H + h
                lo = layer * 512 + h * _C
                den = l_ref[:, c:c + 1] + 1e-16
                val = acc_ref[:, lo:lo + _C] / den + bias[:, lo:lo + _C]
                o_ref[:, lo:lo + _C] = jnp.maximum(val, 0.0)


def _gat(sb, pos8, psq_col, psq_row, b_col, b_row, esed, esedt, esmax, hp,
         biasc):
    def _src(i, j, s):
        return jnp.clip(j, s[0, i], s[1, i])

    grid = (_NDB, _NSB)
    return pl.pallas_call(
        _gat_body,
        grid_spec=pltpu.PrefetchScalarGridSpec(
            num_scalar_prefetch=1,
            grid=grid,
            in_specs=[
                pl.BlockSpec((_BM, 8), lambda i, j, s: (i, 0)),
                pl.BlockSpec((_BS, 8), lambda i, j, s: (_src(i, j, s), 0)),
                pl.BlockSpec((_BM, 1), lambda i, j, s: (i, 0)),
                pl.BlockSpec((1, _BS), lambda i, j, s: (0, _src(i, j, s))),
                pl.BlockSpec((_BM, 1), lambda i, j, s: (i, 0)),
                pl.BlockSpec((1, _BS), lambda i, j, s: (0, _src(i, j, s))),
                pl.BlockSpec((_BM, 32), lambda i, j, s: (i, 0)),
                pl.BlockSpec((32, _BS), lambda i, j, s: (0, _src(i, j, s))),
                pl.BlockSpec((1, 32), lambda i, j, s: (0, 0)),
                pl.BlockSpec((_BS, 1024), lambda i, j, s: (_src(i, j, s), 0)),
                pl.BlockSpec((1, 1024), lambda i, j, s: (0, 0)),
            ],
            out_specs=pl.BlockSpec((_BM, 1024), lambda i, j, s: (i, 0)),
            scratch_shapes=[
                pltpu.VMEM((_BM, 16), jnp.float32),
                pltpu.VMEM((_BM, 16), jnp.float32),
                pltpu.VMEM((_BM, 1024), jnp.float32),
            ],
        ),
        out_shape=jax.ShapeDtypeStruct((_MP, 1024), jnp.float32),
        compiler_params=pltpu.CompilerParams(
            dimension_semantics=("arbitrary", "arbitrary")),
    )(sb, pos8, pos8, psq_col, psq_row, b_col, b_row, esed, esedt, esmax, hp,
      biasc)


def _lstm_body(xq_ref, hin_ref, wih_ref, whh_ref, bias_ref, c_ref,
               h_out_ref, c_out_ref, acc_ref, *, nkx, nkh):
    k = pl.program_id(0)

    @pl.when(k == 0)
    def _():
        acc_ref[...] = jnp.zeros_like(acc_ref[...])

    @pl.when(k < nkx)
    def _():
        acc_ref[...] = acc_ref[...] + jax.lax.dot_general(
            xq_ref[...], wih_ref[...], (((1,), (1,)), ((), ())),
            preferred_element_type=jnp.float32)

    @pl.when(k >= nkx)
    def _():
        acc_ref[...] = acc_ref[...] + jax.lax.dot_general(
            hin_ref[...], whh_ref[...], (((1,), (1,)), ((), ())),
            preferred_element_type=jnp.float32)

    @pl.when(k == nkx + nkh - 1)
    def _():
        g = acc_ref[...] + bias_ref[...]
        gi = g[:, 0:1024]
        gf = g[:, 1024:2048]
        gg = g[:, 2048:3072]
        go = g[:, 3072:4096]
        c_new = jax.nn.sigmoid(gf) * c_ref[...] + jax.nn.sigmoid(gi) * jnp.tanh(gg)
        h_out_ref[...] = jax.nn.sigmoid(go) * jnp.tanh(c_new)
        c_out_ref[...] = c_new


def _lstm(xq, hin, wih, whh, bias, c_prev):
    bk = 512
    nkx = xq.shape[1] // bk
    nkh = hin.shape[1] // bk
    grid = (nkx + nkh,)
    return pl.pallas_call(
        functools.partial(_lstm_body, nkx=nkx, nkh=nkh),
        grid=grid,
        in_specs=[
            pl.BlockSpec((16, bk), lambda k: (0, jnp.minimum(k, nkx - 1))),
            pl.BlockSpec((16, bk),
                         lambda k: (0, jnp.clip(k - nkx, 0, nkh - 1))),
            pl.BlockSpec((4096, bk), lambda k: (0, jnp.minimum(k, nkx - 1))),
            pl.BlockSpec((4096, bk),
                         lambda k: (0, jnp.clip(k - nkx, 0, nkh - 1))),
            pl.BlockSpec((1, 4096), lambda k: (0, 0)),
            pl.BlockSpec((16, 1024), lambda k: (0, 0)),
        ],
        out_specs=[
            pl.BlockSpec((16, 1024), lambda k: (0, 0)),
            pl.BlockSpec((16, 1024), lambda k: (0, 0)),
        ],
        out_shape=[
            jax.ShapeDtypeStruct((16, 1024), jnp.float32),
            jax.ShapeDtypeStruct((16, 1024), jnp.float32),
        ],
        scratch_shapes=[pltpu.VMEM((16, 4096), jnp.float32)],
        compiler_params=pltpu.CompilerParams(
            dimension_semantics=("arbitrary",),
            vmem_limit_bytes=100 * 1024 * 1024),
    )(xq, hin, wih, whh, bias, c_prev)


def _att_body(xc_ref, q_ref, b_ref, s_ref, r_ref, e_scr, mx_scr):
    p = pl.program_id(0)
    b = pl.program_id(1)
    oh = b_ref[...] == jax.lax.broadcasted_iota(jnp.int32, (1, _NB), 1)

    @pl.when((p == 0) & (b == 0))
    def _():
        mx_scr[...] = jnp.full_like(mx_scr[...], _NEG)
        s_ref[...] = jnp.zeros_like(s_ref[...])
        r_ref[...] = jnp.zeros_like(r_ref[...])

    @pl.when(p == 0)
    def _():
        e_full = jax.lax.dot_general(
            xc_ref[...], q_ref[...], (((1,), (1,)), ((), ())),
            preferred_element_type=jnp.float32)
        e = jnp.sum(jnp.where(oh, e_full, 0.0), axis=1, keepdims=True)
        e_scr[pl.ds(b * _BSA, _BSA), :] = e
        mx_scr[...] = jnp.maximum(
            mx_scr[...],
            jnp.max(jnp.where(oh, e_full, _NEG), axis=0, keepdims=True))

    @pl.when(p == 1)
    def _():
        e = e_scr[pl.ds(b * _BSA, _BSA), :]
        mxg = jnp.sum(jnp.where(oh, mx_scr[...], 0.0), axis=1, keepdims=True)
        ee = jnp.exp(e - mxg)
        wee = jnp.where(oh, ee, 0.0)
        s_ref[...] = s_ref[...] + jnp.sum(wee, axis=0, keepdims=True)
        r_ref[...] = r_ref[...] + jax.lax.dot_general(
            wee, xc_ref[...], (((0,), (0,)), ((), ())),
            preferred_element_type=jnp.float32)


def _att(xc, q, b_col):
    grid = (2, _MP // _BSA)
    return pl.pallas_call(
        _att_body,
        grid=grid,
        in_specs=[
            pl.BlockSpec((_BSA, 1024), lambda p, b: (b, 0)),
            pl.BlockSpec((16, 1024), lambda p, b: (0, 0)),
            pl.BlockSpec((_BSA, 1), lambda p, b: (b, 0)),
        ],
        out_specs=[
            pl.BlockSpec((1, _NB), lambda p, b: (0, 0)),
            pl.BlockSpec((_NB, 1024), lambda p, b: (0, 0)),
        ],
        out_shape=[
            jax.ShapeDtypeStruct((1, _NB), jnp.float32),
            jax.ShapeDtypeStruct((_NB, 1024), jnp.float32),
        ],
        scratch_shapes=[
            pltpu.VMEM((_MP, 1), jnp.float32),
            pltpu.VMEM((1, _NB), jnp.float32),
        ],
        compiler_params=pltpu.CompilerParams(
            dimension_semantics=("arbitrary", "arbitrary")),
    )(xc, q, b_col)


def _head_body(xs_ref, w2_ref, g2_ref, b2_ref, w3_ref, b3_ref, o_ref,
               acc_ref):
    k = pl.program_id(0)
    nk = pl.num_programs(0)

    @pl.when(k == 0)
    def _():
        acc_ref[...] = jnp.zeros_like(acc_ref[...])

    acc_ref[...] = acc_ref[...] + jax.lax.dot_general(
        xs_ref[...], w2_ref[...], (((1,), (1,)), ((), ())),
        preferred_element_type=jnp.float32)

    @pl.when(k == nk - 1)
    def _():
        xl = jnp.maximum(acc_ref[...], 0.0) * _BN_SCALE
        xl = xl * g2_ref[...] + b2_ref[...]
        o_ref[...] = jax.lax.dot_general(
            xl, w3_ref[...], (((1,), (1,)), ((), ())),
            preferred_element_type=jnp.float32) + b3_ref[...]


def _head(xs, w2, g2, b2, w3, b3):
    bk = 512
    grid = (2048 // bk,)
    return pl.pallas_call(
        _head_body,
        grid=grid,
        in_specs=[
            pl.BlockSpec((16, bk), lambda k: (0, k)),
            pl.BlockSpec((1024, bk), lambda k: (0, k)),
            pl.BlockSpec((1, 1024), lambda k: (0, 0)),
            pl.BlockSpec((1, 1024), lambda k: (0, 0)),
            pl.BlockSpec((10, 1024), lambda k: (0, 0)),
            pl.BlockSpec((1, 10), lambda k: (0, 0)),
        ],
        out_specs=pl.BlockSpec((16, 10), lambda k: (0, 0)),
        out_shape=jax.ShapeDtypeStruct((16, 10), jnp.float32),
        scratch_shapes=[pltpu.VMEM((16, 1024), jnp.float32)],
        compiler_params=pltpu.CompilerParams(
            dimension_semantics=("arbitrary",)),
    )(xs, w2, g2.reshape(1, -1), b2.reshape(1, -1), w3, b3.reshape(1, -1))


def _amat(c1_as, c1_ad, c2_as, c2_ad):
    amat = jnp.zeros((1024, 32), jnp.float32)
    rows = jnp.arange(_H * _C)
    head = rows // _C
    amat = amat.at[rows, head].set(c1_as.reshape(-1))
    amat = amat.at[rows, 8 + head].set(c1_ad.reshape(-1))
    amat = amat.at[512 + rows, 16 + head].set(c2_as.reshape(-1))
    amat = amat.at[512 + rows, 24 + head].set(c2_ad.reshape(-1))
    return amat


@jax.jit
def kernel(x, pos, batch, lin01_w, bn1_g, bn1_b, c1_W, c1_as, c1_ad, c1_b,
           c2_W, c2_as, c2_ad, c2_b, Wih0, Whh0, bih0, bhh0, Wih1, Whh1,
           bih1, bhh1, lin2_w, bn2_g, bn2_b, lin3_w, lin3_b):
    n = x.shape[0]
    pad = _MP - n

    x_pad = jnp.pad(x, ((0, pad), (0, 0)))
    pos8 = jnp.pad(pos.astype(jnp.float32), ((0, pad), (0, 5)))
    batch_pad = jnp.concatenate(
        [batch.astype(jnp.int32), jnp.full((pad,), 127, jnp.int32)])
    psq = jnp.sum(pos8 * pos8, axis=1)
    psq_col = psq.reshape(_MP, 1)
    psq_row = psq.reshape(1, _MP)
    b_col = batch_pad.reshape(_MP, 1)
    b_row = batch_pad.reshape(1, _MP)

    row0 = jnp.arange(_NDB, dtype=jnp.int32) * _BM
    g_lo = batch_pad[row0]
    g_hi = batch_pad[row0 + _BM - 1]
    start_row = jnp.searchsorted(batch_pad, g_lo, side='left')
    end_row = jnp.searchsorted(batch_pad, g_hi, side='right')
    sb = jnp.stack([start_row // _BS, (end_row - 1) // _BS]).astype(jnp.int32)

    h = _mm1(x_pad, lin01_w, bn1_g, bn1_b)

    wc = jnp.concatenate([c1_W, c2_W], axis=1)
    amat = _amat(c1_as, c1_ad, c2_as, c2_ad)
    hp, esed, esmax = _proj(h, wc, amat)
    esedt = esed.T

    biasc = jnp.concatenate([c1_b, c2_b]).reshape(1, 1024)
    xc = _gat(sb, pos8, psq_col, psq_row, b_col, b_row, esed, esedt, esmax,
              hp, biasc)

    bias0 = (bih0 + bhh0).reshape(1, 4096)
    bias1 = (bih1 + bhh1).reshape(1, 4096)
    q_star = jnp.zeros((_NB, 2048), jnp.float32)
    h0 = jnp.zeros((_NB, 1024), jnp.float32)
    c0 = h0
    h1 = h0
    c1 = h0
    for _ in range(2):
        h0, c0 = _lstm(q_star, h0, Wih0, Whh0, bias0, c0)
        h1, c1 = _lstm(h0, h1, Wih1, Whh1, bias1, c1)
        s, r = _att(xc, h1, b_col)
        r = r / (s.reshape(_NB, 1) + 1e-16)
        q_star = jnp.concatenate([h1, r], axis=1)

    return _head(q_star, lin2_w, bn2_g, bn2_b, lin3_w, lin3_b)
